# Initial kernel scaffold; baseline (speedup 1.0000x reference)
import jax
import jax.numpy as jnp
from jax.experimental import pallas as pl


def kernel(x, adj, enc_W1, enc_b1, enc_W2, enc_b2, b1_W, b1_hop, b1_resW, b1_resb, b1_muW, b1_lvW, b2_W, b2_hop, b2_resW, b2_resb, b2_muW, b2_lvW, dec_W, cluster):
    raise NotImplementedError("write your pallas kernel here")


# speedup vs baseline: 1.9708x; 1.9708x over previous
import jax
import jax.numpy as jnp
from jax.experimental import pallas as pl
from jax.experimental.pallas import tpu as pltpu

_N = 2048
_BN_INV = 0.9995003746877732


def _dot(a, b):
    return jax.lax.dot_general(a, b, (((1,), (0,)), ((), ())),
                               preferred_element_type=jnp.float32)


def _elu(v):
    return jnp.where(v > 0, v, jnp.exp(jnp.minimum(v, 0.0)) - 1.0)


def _body(x_ref, adj_ref, encW1_ref, encb1_ref, encW2_ref, encb2_ref,
          b1W_ref, b1hop_ref, b1resW_ref, b1resb_ref, b1muW_ref,
          b2W_ref, b2hop_ref, b2resW_ref, b2resb_ref, b2muW_ref,
          decW_ref, cluster_ref,
          de_ref, q_ref, z_ref, loss_ref):
    adj = adj_ref[...]

    h = _elu((_dot(x_ref[...], encW1_ref[...]) + encb1_ref[...]) * _BN_INV)
    feat = _elu((_dot(h, encW2_ref[...]) + encb2_ref[...]) * _BN_INV)

    def hop_w(hop_ref):
        l = hop_ref[...]
        e = jnp.exp(l - jnp.max(l, axis=1, keepdims=True))
        w = e / jnp.sum(e, axis=1, keepdims=True)
        return w[0, 0], w[0, 1]

    def multi_hop(inp, W_ref, hop_ref, resW_ref, resb_ref):
        w0, w1 = hop_w(hop_ref)
        residual = _dot(inp, resW_ref[...]) + resb_ref[...]
        t1 = _dot(adj, _dot(inp, W_ref[...]))
        t2 = _dot(adj, t1)
        return jnp.maximum(w0 * t1 + w1 * t2 + residual, 0.0)

    hidden1 = multi_hop(feat, b1W_ref, b1hop_ref, b1resW_ref, b1resb_ref)
    mu1 = _dot(adj, _dot(hidden1, b1muW_ref[...]))
    hidden2 = multi_hop(mu1, b2W_ref, b2hop_ref, b2resW_ref, b2resb_ref)
    mu2 = _dot(adj, _dot(hidden2, b2muW_ref[...]))

    z = jnp.concatenate([feat, mu1, mu2], axis=1)
    z_ref[...] = z
    de_ref[...] = _dot(adj, _dot(z, decW_ref[...]))

    eps = 1e-8
    num = jnp.sum(mu1 * mu2, axis=1, keepdims=True)
    n1 = jnp.maximum(jnp.sqrt(jnp.sum(mu1 * mu1, axis=1, keepdims=True)), eps)
    n2 = jnp.maximum(jnp.sqrt(jnp.sum(mu2 * mu2, axis=1, keepdims=True)), eps)
    loss_ref[...] = -jnp.sum(num / (n1 * n2), keepdims=True) * (1.0 / _N)

    c = cluster_ref[...]
    z2 = jnp.sum(z * z, axis=1, keepdims=True)
    c2 = jnp.sum(c * c, axis=1, keepdims=True)
    cross = jax.lax.dot_general(z, c, (((1,), (1,)), ((), ())),
                                preferred_element_type=jnp.float32)
    dist = z2 + jnp.transpose(c2) - 2.0 * cross
    q = 1.0 / (1.0 + dist)
    q_ref[...] = q / jnp.sum(q, axis=1, keepdims=True)


def kernel(x, adj, enc_W1, enc_b1, enc_W2, enc_b2,
           b1_W, b1_hop, b1_resW, b1_resb, b1_muW, b1_lvW,
           b2_W, b2_hop, b2_resW, b2_resb, b2_muW, b2_lvW,
           dec_W, cluster):
    n, d = x.shape
    latent = enc_W2.shape[1] + b1_muW.shape[1] + b2_muW.shape[1]
    k = cluster.shape[0]
    row = lambda v: v.reshape(1, -1)
    de_feat, q, z, loss = pl.pallas_call(
        _body,
        out_shape=(
            jax.ShapeDtypeStruct((n, d), jnp.float32),
            jax.ShapeDtypeStruct((n, k), jnp.float32),
            jax.ShapeDtypeStruct((n, latent), jnp.float32),
            jax.ShapeDtypeStruct((1, 1), jnp.float32),
        ),
        compiler_params=pltpu.CompilerParams(
            vmem_limit_bytes=110 * 1024 * 1024),
    )(x, adj, enc_W1, row(enc_b1), enc_W2, row(enc_b2),
      b1_W, row(b1_hop), b1_resW, row(b1_resb), b1_muW,
      b2_W, row(b2_hop), b2_resW, row(b2_resb), b2_muW,
      dec_W, cluster)
    return (de_feat, q, z, loss[0, 0])


# speedup vs baseline: 1.9791x; 1.0042x over previous
import jax
import jax.numpy as jnp
from jax.experimental import pallas as pl
from jax.experimental.pallas import tpu as pltpu

_N = 2048
_BN_INV = 0.9995003746877732


def _dot(a, b):
    return jax.lax.dot_general(a, b, (((1,), (0,)), ((), ())),
                               preferred_element_type=jnp.float32)


def _dot16(a, b):
    return jax.lax.dot_general(a.astype(jnp.bfloat16), b.astype(jnp.bfloat16),
                               (((1,), (0,)), ((), ())),
                               preferred_element_type=jnp.float32)


def _elu(v):
    return jnp.where(v > 0, v, jnp.exp(jnp.minimum(v, 0.0)) - 1.0)


def _body(x_ref, adj_ref, encW1_ref, encb1_ref, encW2_ref, encb2_ref,
          b1W_ref, b1hop_ref, b1resW_ref, b1resb_ref, b1muW_ref,
          b2W_ref, b2hop_ref, b2resW_ref, b2resb_ref, b2muW_ref,
          decW_ref, cluster_ref,
          de_ref, q_ref, z_ref, loss_ref):
    adj = adj_ref[...].astype(jnp.bfloat16)

    h = _elu((_dot(x_ref[...], encW1_ref[...]) + encb1_ref[...]) * _BN_INV)
    feat = _elu((_dot(h, encW2_ref[...]) + encb2_ref[...]) * _BN_INV)

    def hop_w(hop_ref):
        l = hop_ref[...]
        e = jnp.exp(l - jnp.max(l, axis=1, keepdims=True))
        w = e / jnp.sum(e, axis=1, keepdims=True)
        return w[0, 0], w[0, 1]

    def multi_hop(inp, W_ref, hop_ref, resW_ref, resb_ref):
        w0, w1 = hop_w(hop_ref)
        residual = _dot(inp, resW_ref[...]) + resb_ref[...]
        t1 = _dot16(adj, _dot(inp, W_ref[...]))
        t2 = _dot16(adj, t1)
        return jnp.maximum(w0 * t1 + w1 * t2 + residual, 0.0)

    hidden1 = multi_hop(feat, b1W_ref, b1hop_ref, b1resW_ref, b1resb_ref)
    mu1 = _dot16(adj, _dot(hidden1, b1muW_ref[...]))
    hidden2 = multi_hop(mu1, b2W_ref, b2hop_ref, b2resW_ref, b2resb_ref)
    mu2 = _dot16(adj, _dot(hidden2, b2muW_ref[...]))

    z = jnp.concatenate([feat, mu1, mu2], axis=1)
    z_ref[...] = z
    de_ref[...] = _dot16(adj, _dot(z, decW_ref[...]))

    eps = 1e-8
    num = jnp.sum(mu1 * mu2, axis=1, keepdims=True)
    n1 = jnp.maximum(jnp.sqrt(jnp.sum(mu1 * mu1, axis=1, keepdims=True)), eps)
    n2 = jnp.maximum(jnp.sqrt(jnp.sum(mu2 * mu2, axis=1, keepdims=True)), eps)
    loss_ref[...] = -jnp.sum(num / (n1 * n2), keepdims=True) * (1.0 / _N)

    c = cluster_ref[...]
    z2 = jnp.sum(z * z, axis=1, keepdims=True)
    c2 = jnp.sum(c * c, axis=1, keepdims=True)
    cross = jax.lax.dot_general(z, c, (((1,), (1,)), ((), ())),
                                preferred_element_type=jnp.float32)
    dist = z2 + jnp.transpose(c2) - 2.0 * cross
    q = 1.0 / (1.0 + dist)
    q_ref[...] = q / jnp.sum(q, axis=1, keepdims=True)


def kernel(x, adj, enc_W1, enc_b1, enc_W2, enc_b2,
           b1_W, b1_hop, b1_resW, b1_resb, b1_muW, b1_lvW,
           b2_W, b2_hop, b2_resW, b2_resb, b2_muW, b2_lvW,
           dec_W, cluster):
    n, d = x.shape
    latent = enc_W2.shape[1] + b1_muW.shape[1] + b2_muW.shape[1]
    k = cluster.shape[0]
    row = lambda v: v.reshape(1, -1)
    de_feat, q, z, loss = pl.pallas_call(
        _body,
        out_shape=(
            jax.ShapeDtypeStruct((n, d), jnp.float32),
            jax.ShapeDtypeStruct((n, k), jnp.float32),
            jax.ShapeDtypeStruct((n, latent), jnp.float32),
            jax.ShapeDtypeStruct((1, 1), jnp.float32),
        ),
        compiler_params=pltpu.CompilerParams(
            vmem_limit_bytes=110 * 1024 * 1024),
    )(x, adj, enc_W1, row(enc_b1), enc_W2, row(enc_b2),
      b1_W, row(b1_hop), b1_resW, row(b1_resb), b1_muW,
      b2_W, row(b2_hop), b2_resW, row(b2_resb), b2_muW,
      dec_W, cluster)
    return (de_feat, q, z, loss[0, 0])
